# baseline (speedup 1.0000x reference)
import functools

import jax
import jax.numpy as jnp
from jax import lax
from jax.experimental import pallas as pl
from jax.experimental.pallas import tpu as pltpu
from jax.experimental.pallas import tpu_sc as plsc

DENSE = 1024
BATCH = 1024
SPIKES = 64
NC, NS, L = 2, 16, 16
NW = NC * NS
BPW = BATCH // NW


def _build_counts_body(ids_hbm, ns_hbm, a_hbm, a_v, ids_v, ns_v):
    wid = lax.axis_index("s") * NC + lax.axis_index("c")
    base = wid * BPW
    pltpu.sync_copy(ids_hbm.at[pl.ds(base, BPW)], ids_v)
    pltpu.sync_copy(ns_hbm.at[pl.ds(base, BPW)], ns_v)

    zeros = jnp.zeros((L,), jnp.float32)
    ones = jnp.ones((L,), jnp.float32)
    lanes = lax.iota(jnp.int32, L)

    def zero_body(b, carry):
        for k in range(DENSE // L):
            a_v[b, pl.ds(k * L, L)] = zeros
        return carry

    lax.fori_loop(0, BPW, zero_body, 0)

    for g in range(BPW // L):
        ns16 = ns_v[pl.ds(g * L, L)]
        bvec = lanes + (g * L)

        def spike_body(j, carry, ns16=ns16, bvec=bvec):
            ids16 = plsc.load_gather(ids_v, [bvec, lax.broadcast(j, (L,))])
            mask = ns16 > j
            plsc.addupdate_scatter(a_v, [bvec, ids16], ones, mask=mask)
            return carry

        lax.fori_loop(0, SPIKES, spike_body, 0)
    pltpu.sync_copy(a_v, a_hbm.at[pl.ds(base, BPW)])


_build_counts = functools.partial(
    pl.kernel,
    out_type=jax.ShapeDtypeStruct((BATCH, DENSE), jnp.float32),
    mesh=plsc.VectorSubcoreMesh(core_axis_name="c", subcore_axis_name="s"),
    compiler_params=pltpu.CompilerParams(
        needs_layout_passes=False, skip_device_barrier=True),
    scratch_types=[
        pltpu.VMEM((BPW, DENSE), jnp.float32),
        pltpu.VMEM((BPW, SPIKES), jnp.int32),
        pltpu.VMEM((BPW,), jnp.int32),
    ],
)(_build_counts_body)


def _matmul_body(a_ref, w_ref, o_ref):
    o_ref[...] = lax.dot_general(
        a_ref[...].astype(jnp.bfloat16), w_ref[...],
        dimension_numbers=(((1,), (1,)), ((), ())),
        preferred_element_type=jnp.float32,
    )


def _matmul(a, w):
    nblk = 8
    return pl.pallas_call(
        _matmul_body,
        grid=(nblk,),
        in_specs=[
            pl.BlockSpec((BATCH // nblk, DENSE), lambda i: (i, 0)),
            pl.BlockSpec((DENSE, DENSE), lambda i: (0, 0)),
        ],
        out_specs=pl.BlockSpec((BATCH // nblk, DENSE), lambda i: (i, 0)),
        out_shape=jax.ShapeDtypeStruct((BATCH, DENSE), jnp.float32),
    )(a, w)


def kernel(w, spike_ids, num_spikes):
    a = _build_counts(spike_ids, num_spikes)
    return _matmul(a, w.astype(jnp.bfloat16))


# speedup vs baseline: 1.1126x; 1.1126x over previous
import functools

import jax
import jax.numpy as jnp
from jax import lax
from jax.experimental import pallas as pl
from jax.experimental.pallas import tpu as pltpu
from jax.experimental.pallas import tpu_sc as plsc

DENSE = 1024
BATCH = 1024
SPIKES = 64
NC, NS, L = 2, 16, 16
NW = NC * NS
BPW = BATCH // NW


def _build_counts_body(ids_hbm, ns_hbm, a_hbm, a_v, ids_v, ns_v):
    wid = lax.axis_index("s") * NC + lax.axis_index("c")
    base = wid * BPW
    pltpu.sync_copy(ids_hbm.at[pl.ds(base, BPW)], ids_v)
    pltpu.sync_copy(ns_hbm.at[pl.ds(base, BPW)], ns_v)

    zeros = jnp.zeros((L,), jnp.float32)
    ones = jnp.ones((L,), jnp.float32)
    lanes = lax.iota(jnp.int32, L)

    def zero_body(b, carry):
        for k in range(DENSE // L):
            a_v[b, pl.ds(k * L, L)] = zeros
        return carry

    lax.fori_loop(0, BPW, zero_body, 0)

    for g in range(BPW // L):
        ns16 = ns_v[pl.ds(g * L, L)]
        bvec = lanes + (g * L)

        def spike_body(j, carry, ns16=ns16, bvec=bvec):
            ids16 = plsc.load_gather(ids_v, [bvec, lax.broadcast(j, (L,))])
            mask = ns16 > j
            plsc.addupdate_scatter(a_v, [bvec, ids16], ones, mask=mask)
            return carry

        lax.fori_loop(0, SPIKES, spike_body, 0)
    pltpu.sync_copy(a_v, a_hbm.at[pl.ds(base, BPW)])


_build_counts = functools.partial(
    pl.kernel,
    out_type=jax.ShapeDtypeStruct((BATCH, DENSE), jnp.float32),
    mesh=plsc.VectorSubcoreMesh(core_axis_name="c", subcore_axis_name="s"),
    compiler_params=pltpu.CompilerParams(
        needs_layout_passes=False, skip_device_barrier=True),
    scratch_types=[
        pltpu.VMEM((BPW, DENSE), jnp.float32),
        pltpu.VMEM((BPW, SPIKES), jnp.int32),
        pltpu.VMEM((BPW,), jnp.int32),
    ],
)(_build_counts_body)


def _matmul_body(a_ref, w_ref, o_ref):
    o_ref[...] = lax.dot_general(
        a_ref[...].astype(jnp.bfloat16), w_ref[...],
        dimension_numbers=(((1,), (1,)), ((), ())),
        preferred_element_type=jnp.float32,
    )


def _matmul(a, w):
    nblk = 4
    return pl.pallas_call(
        _matmul_body,
        grid=(nblk,),
        in_specs=[
            pl.BlockSpec((BATCH // nblk, DENSE), lambda i: (i, 0)),
            pl.BlockSpec((DENSE, DENSE), lambda i: (0, 0)),
        ],
        out_specs=pl.BlockSpec((BATCH // nblk, DENSE), lambda i: (i, 0)),
        out_shape=jax.ShapeDtypeStruct((BATCH, DENSE), jnp.float32),
    )(a, w)


def kernel(w, spike_ids, num_spikes):
    a = _build_counts(spike_ids, num_spikes)
    return _matmul(a, w.astype(jnp.bfloat16))


# speedup vs baseline: 1.1666x; 1.0485x over previous
import functools

import jax
import jax.numpy as jnp
from jax import lax
from jax.experimental import pallas as pl
from jax.experimental.pallas import tpu as pltpu
from jax.experimental.pallas import tpu_sc as plsc

DENSE = 1024
BATCH = 1024
SPIKES = 64
NC, NS, L = 2, 16, 16
NW = NC * NS
BPW = BATCH // NW


def _build_counts_body(ids_hbm, ns_hbm, a_hbm, a_v, ids_v, ns_v, sem, sem_in):
    wid = lax.axis_index("s") * NC + lax.axis_index("c")
    base = wid * BPW
    cp_ids = pltpu.async_copy(ids_hbm.at[pl.ds(base, BPW)], ids_v, sem_in)
    cp_ns = pltpu.async_copy(ns_hbm.at[pl.ds(base, BPW)], ns_v, sem_in)

    zeros = jnp.zeros((L,), jnp.float32)
    ones = jnp.ones((L,), jnp.float32)
    lanes = lax.iota(jnp.int32, L)

    def zero_body(b, carry):
        for k in range(DENSE // L):
            a_v[b, pl.ds(k * L, L)] = zeros
        return carry

    lax.fori_loop(0, L, zero_body, 0)
    cp_ids.wait()
    cp_ns.wait()

    copies = []
    for g in range(BPW // L):
        if g > 0:
            lax.fori_loop(g * L, (g + 1) * L, zero_body, 0)

        ns16 = ns_v[pl.ds(g * L, L)]
        bvec = lanes + (g * L)

        def spike_body(j, carry, ns16=ns16, bvec=bvec):
            ids16 = plsc.load_gather(ids_v, [bvec, lax.broadcast(j, (L,))])
            mask = ns16 > j
            plsc.addupdate_scatter(a_v, [bvec, ids16], ones, mask=mask)
            return carry

        lax.fori_loop(0, SPIKES, spike_body, 0)
        copies.append(pltpu.async_copy(
            a_v.at[pl.ds(g * L, L)],
            a_hbm.at[pl.ds(base + g * L, L)], sem))
    for cp in copies:
        cp.wait()


_build_counts = functools.partial(
    pl.kernel,
    out_type=jax.ShapeDtypeStruct((BATCH, DENSE), jnp.float32),
    mesh=plsc.VectorSubcoreMesh(core_axis_name="c", subcore_axis_name="s"),
    compiler_params=pltpu.CompilerParams(
        needs_layout_passes=False, skip_device_barrier=True,
        disable_bounds_checks=True, disable_semaphore_checks=True),
    scratch_types=[
        pltpu.VMEM((BPW, DENSE), jnp.float32),
        pltpu.VMEM((BPW, SPIKES), jnp.int32),
        pltpu.VMEM((BPW,), jnp.int32),
        pltpu.SemaphoreType.DMA,
        pltpu.SemaphoreType.DMA,
    ],
)(_build_counts_body)


def _matmul_body(a_ref, w_ref, o_ref):
    o_ref[...] = lax.dot_general(
        a_ref[...].astype(jnp.bfloat16), w_ref[...],
        dimension_numbers=(((1,), (1,)), ((), ())),
        preferred_element_type=jnp.float32,
    )


def _matmul(a, w):
    nblk = 4
    return pl.pallas_call(
        _matmul_body,
        grid=(nblk,),
        in_specs=[
            pl.BlockSpec((BATCH // nblk, DENSE), lambda i: (i, 0)),
            pl.BlockSpec((DENSE, DENSE), lambda i: (0, 0)),
        ],
        out_specs=pl.BlockSpec((BATCH // nblk, DENSE), lambda i: (i, 0)),
        out_shape=jax.ShapeDtypeStruct((BATCH, DENSE), jnp.float32),
    )(a, w)


def kernel(w, spike_ids, num_spikes):
    a = _build_counts(spike_ids, num_spikes)
    return _matmul(a, w.astype(jnp.bfloat16))


# speedup vs baseline: 1.1912x; 1.0211x over previous
import functools

import jax
import jax.numpy as jnp
from jax import lax
from jax.experimental import pallas as pl
from jax.experimental.pallas import tpu as pltpu
from jax.experimental.pallas import tpu_sc as plsc

DENSE = 1024
BATCH = 1024
SPIKES = 64
NC, NS, L = 2, 16, 16
NW = NC * NS
BPW = BATCH // NW


def _build_counts_body(ids_hbm, ns_hbm, a_hbm, a_v, ids_v, ns_v, sem, sem_in):
    wid = lax.axis_index("s") * NC + lax.axis_index("c")
    base = wid * BPW
    cp_ids = pltpu.async_copy(ids_hbm.at[pl.ds(base, BPW)], ids_v, sem_in)
    cp_ns = pltpu.async_copy(ns_hbm.at[pl.ds(base, BPW)], ns_v, sem_in)

    zeros = jnp.zeros((L,), jnp.float32)
    ones = jnp.ones((L,), jnp.float32)
    lanes = lax.iota(jnp.int32, L)

    def zero_body(b, carry):
        for k in range(DENSE // L):
            a_v[b, pl.ds(k * L, L)] = zeros
        return carry

    lax.fori_loop(0, L, zero_body, 0)
    cp_ids.wait()
    cp_ns.wait()

    copies = []
    for g in range(BPW // L):
        if g > 0:
            lax.fori_loop(g * L, (g + 1) * L, zero_body, 0)

        ns16 = ns_v[pl.ds(g * L, L)]
        bvec = lanes + (g * L)

        def spike_body(j, carry, ns16=ns16, bvec=bvec):
            ids16 = plsc.load_gather(ids_v, [bvec, lax.broadcast(j, (L,))])
            mask = ns16 > j
            plsc.addupdate_scatter(a_v, [bvec, ids16], ones, mask=mask)
            return carry

        lax.fori_loop(0, SPIKES, spike_body, 0)
        copies.append(pltpu.async_copy(
            a_v.at[pl.ds(g * L, L)],
            a_hbm.at[pl.ds(base + g * L, L)], sem))
    for cp in copies:
        cp.wait()


_build_counts = functools.partial(
    pl.kernel,
    out_type=jax.ShapeDtypeStruct((BATCH, DENSE), jnp.float32),
    mesh=plsc.VectorSubcoreMesh(core_axis_name="c", subcore_axis_name="s"),
    compiler_params=pltpu.CompilerParams(
        needs_layout_passes=False, skip_device_barrier=True,
        disable_bounds_checks=True, disable_semaphore_checks=True),
    scratch_types=[
        pltpu.VMEM((BPW, DENSE), jnp.float32),
        pltpu.VMEM((BPW, SPIKES), jnp.int32),
        pltpu.VMEM((BPW,), jnp.int32),
        pltpu.SemaphoreType.DMA,
        pltpu.SemaphoreType.DMA,
    ],
)(_build_counts_body)


def _matmul_body(a_ref, w_ref, o_ref):
    o_ref[...] = lax.dot_general(
        a_ref[...].astype(jnp.bfloat16), w_ref[...],
        dimension_numbers=(((1,), (1,)), ((), ())),
        preferred_element_type=jnp.float32,
    )


def _matmul(a, w):
    nblk = 2
    return pl.pallas_call(
        _matmul_body,
        grid=(nblk,),
        in_specs=[
            pl.BlockSpec((BATCH // nblk, DENSE), lambda i: (i, 0)),
            pl.BlockSpec((DENSE, DENSE), lambda i: (0, 0)),
        ],
        out_specs=pl.BlockSpec((BATCH // nblk, DENSE), lambda i: (i, 0)),
        out_shape=jax.ShapeDtypeStruct((BATCH, DENSE), jnp.float32),
    )(a, w)


def kernel(w, spike_ids, num_spikes):
    a = _build_counts(spike_ids, num_spikes)
    return _matmul(a, w.astype(jnp.bfloat16))
